# baseline (speedup 1.0000x reference)
import functools

import jax
import jax.numpy as jnp
from jax import lax
from jax.experimental import pallas as pl
from jax.experimental.pallas import tpu as pltpu
from jax.experimental.pallas import tpu_sc as plsc

_E = 64
_TEMP = 0.7
_P_MIN = 0.92
_K = 3
_BAL_W = 0.01
_BLK = 1024
_N = 16384
_NW = 32
_CHUNKS = (8192, 4096, 4096)


def _mlp_body(x_ref, w1_ref, w2_ref, o_ref):
    h = jnp.tanh(lax.dot_general(
        x_ref[...], w1_ref[...], (((1,), (1,)), ((), ())),
        preferred_element_type=jnp.float32))
    o_ref[0] = lax.dot_general(
        w2_ref[...], h, (((1,), (1,)), ((), ())),
        preferred_element_type=jnp.float32) * (1.0 / _TEMP)


def _mlp_logits_t(x, w1, w2, start_blk, nb):
    n, d = x.shape
    return pl.pallas_call(
        _mlp_body,
        grid=(nb,),
        in_specs=[
            pl.BlockSpec((_BLK, d), lambda i: (i + start_blk, 0)),
            pl.BlockSpec((_E, d), lambda i: (0, 0)),
            pl.BlockSpec((_E, _E), lambda i: (0, 0)),
        ],
        out_specs=pl.BlockSpec((1, _E, _BLK), lambda i: (i, 0, 0)),
        out_shape=jax.ShapeDtypeStruct((nb, _E, _BLK), jnp.float32),
    )(x, w1, w2)


def _route_body(nc, lg_hbm, i1_hbm, i2_hbm, i3_hbm, t1_hbm, t2_hbm, t3_hbm,
                m2_hbm, m3_hbm, k_hbm, imp_hbm,
                lg_v, i1_v, i2_v, i3_v, t1_v, t2_v, t3_v,
                m2_v, m3_v, k_v, impl_v):
    c = nc // _NW
    ng = c // 16
    wid = lax.axis_index("s") * 2 + lax.axis_index("c")
    base = wid * c
    blk = wid // (_BLK // c)
    off = (wid % (_BLK // c)) * c
    pltpu.sync_copy(lg_hbm.at[blk, :, pl.ds(off, c)], lg_v)

    zero16 = jnp.zeros((16,), jnp.float32)
    for e in range(_E):
        impl_v[e, :] = zero16

    def group(g, carry):
        tok = g * 16
        neg = jnp.full((16,), -jnp.inf, jnp.float32)
        m1, m2, m3 = neg, neg, neg
        i1 = jnp.zeros((16,), jnp.int32)
        i2, i3 = i1, i1
        for e in range(_E):
            v = lg_v[e, pl.ds(tok, 16)]
            ei = jnp.full((16,), e, jnp.int32)
            c1 = v > m1
            c2 = v > m2
            c3 = v > m3
            m3 = jnp.where(c2, m2, jnp.where(c3, v, m3))
            i3 = jnp.where(c2, i2, jnp.where(c3, ei, i3))
            m2 = jnp.where(c1, m1, jnp.where(c2, v, m2))
            i2 = jnp.where(c1, i1, jnp.where(c2, ei, i2))
            m1 = jnp.where(c1, v, m1)
            i1 = jnp.where(c1, ei, i1)
        acc = jnp.zeros((16,), jnp.float32)
        for e in range(_E):
            acc = acc + jnp.exp(lg_v[e, pl.ds(tok, 16)] - m1)
        p1 = 1.0 / acc
        p2 = jnp.exp(m2 - m1) / acc
        p3 = jnp.exp(m3 - m1) / acc
        k = jnp.where(p1 >= _P_MIN, 1,
                      jnp.where(p1 + p2 >= _P_MIN, 2, 3)).astype(jnp.int32)
        mk2 = jnp.where(k >= 2, 1.0, 0.0)
        mk3 = jnp.where(k >= 3, 1.0, 0.0)
        s = p1 + p2 * mk2 + p3 * mk3
        inv = 1.0 / (s + 1e-9)
        ts1 = p1 * inv
        ts2 = p2 * mk2 * inv
        ts3 = p3 * mk3 * inv
        sl = pl.ds(tok, 16)
        i1_v[sl] = i1
        i2_v[sl] = i2
        i3_v[sl] = i3
        t1_v[sl] = ts1
        t2_v[sl] = ts2
        t3_v[sl] = ts3
        m2_v[sl] = mk2
        m3_v[sl] = mk3
        k_v[sl] = k
        for e in range(_E):
            ei = jnp.full((16,), e, jnp.int32)
            cnt = (jnp.where(i1 == ei, ts1, 0.0)
                   + jnp.where(i2 == ei, ts2, 0.0)
                   + jnp.where(i3 == ei, ts3, 0.0))
            impl_v[e, :] = impl_v[e, :] + cnt
        return carry

    lax.fori_loop(0, ng, group, 0)

    for src, dst in ((i1_v, i1_hbm), (i2_v, i2_hbm), (i3_v, i3_hbm),
                     (t1_v, t1_hbm), (t2_v, t2_hbm), (t3_v, t3_hbm),
                     (m2_v, m2_hbm), (m3_v, m3_hbm), (k_v, k_hbm)):
        pltpu.sync_copy(src, dst.at[pl.ds(base, c)])
    pltpu.sync_copy(impl_v, imp_hbm.at[pl.ds(wid * _E, _E), :])


@functools.cache
def _make_route(nc):
    c = nc // _NW
    mesh = plsc.VectorSubcoreMesh(core_axis_name="c", subcore_axis_name="s")
    i32v = jax.ShapeDtypeStruct((nc,), jnp.int32)
    f32v = jax.ShapeDtypeStruct((nc,), jnp.float32)
    return pl.kernel(
        functools.partial(_route_body, nc),
        mesh=mesh,
        out_type=(i32v, i32v, i32v, f32v, f32v, f32v, f32v, f32v, i32v,
                  jax.ShapeDtypeStruct((_NW * _E, 16), jnp.float32)),
        scratch_types=[
            pltpu.VMEM((_E, c), jnp.float32),
            pltpu.VMEM((c,), jnp.int32),
            pltpu.VMEM((c,), jnp.int32),
            pltpu.VMEM((c,), jnp.int32),
            pltpu.VMEM((c,), jnp.float32),
            pltpu.VMEM((c,), jnp.float32),
            pltpu.VMEM((c,), jnp.float32),
            pltpu.VMEM((c,), jnp.float32),
            pltpu.VMEM((c,), jnp.float32),
            pltpu.VMEM((c,), jnp.int32),
            pltpu.VMEM((_E, 16), jnp.float32),
        ],
    )


def _loss_body(*refs):
    part_refs, loss_ref = refs[:-1], refs[-1]
    imp = jnp.zeros((_E,), jnp.float32)
    for part_ref in part_refs:
        s = jnp.sum(part_ref[...], axis=1)
        imp = imp + jnp.sum(s.reshape(_NW, _E), axis=0)
    mean = jnp.sum(imp) * (1.0 / _E)
    var = jnp.sum((imp - mean) ** 2) * (1.0 / _E)
    loss = _BAL_W * var / (mean * mean + 1e-10)
    loss_ref[...] = loss * jnp.ones((1, 1), jnp.float32)


def _loss(parts):
    return pl.pallas_call(
        _loss_body,
        out_shape=jax.ShapeDtypeStruct((1, 1), jnp.float32),
    )(*parts)


@jax.jit
def kernel(x, W1, W2):
    n, _ = x.shape
    chunks = []
    start = 0
    for nc in _CHUNKS:
        logits_t = _mlp_logits_t(x, W1, W2, start // _BLK, nc // _BLK)
        chunks.append(_make_route(nc)(logits_t))
        start += nc
    i1, i2, i3, t1, t2, t3, m2, m3, k_vec = (
        jnp.concatenate([ch[j] for ch in chunks]) if len(chunks) > 1
        else chunks[0][j] for j in range(9))
    loss = _loss([ch[9] for ch in chunks])
    top_idx = jnp.stack([i1, i2, i3], axis=1)
    top_scores = jnp.stack([t1, t2, t3], axis=1)
    top_mask = jnp.stack([jnp.ones((n,), jnp.float32), m2, m3], axis=1)
    return (top_idx, top_scores, top_mask, k_vec, loss.reshape(()))


# speedup vs baseline: 1.0387x; 1.0387x over previous
import functools

import jax
import jax.numpy as jnp
from jax import lax
from jax.experimental import pallas as pl
from jax.experimental.pallas import tpu as pltpu
from jax.experimental.pallas import tpu_sc as plsc

_E = 64
_TEMP = 0.7
_P_MIN = 0.92
_K = 3
_BAL_W = 0.01
_BLK = 1024
_N = 16384
_NW = 32
_CHUNKS = (8192, 8192)


def _mlp_body(x_ref, w1_ref, w2_ref, o_ref):
    h = jnp.tanh(lax.dot_general(
        x_ref[...], w1_ref[...], (((1,), (1,)), ((), ())),
        preferred_element_type=jnp.float32))
    o_ref[0] = lax.dot_general(
        w2_ref[...], h, (((1,), (1,)), ((), ())),
        preferred_element_type=jnp.float32) * (1.0 / _TEMP)


def _mlp_logits_t(x, w1, w2, start_blk, nb):
    n, d = x.shape
    return pl.pallas_call(
        _mlp_body,
        grid=(nb,),
        in_specs=[
            pl.BlockSpec((_BLK, d), lambda i: (i + start_blk, 0)),
            pl.BlockSpec((_E, d), lambda i: (0, 0)),
            pl.BlockSpec((_E, _E), lambda i: (0, 0)),
        ],
        out_specs=pl.BlockSpec((1, _E, _BLK), lambda i: (i, 0, 0)),
        out_shape=jax.ShapeDtypeStruct((nb, _E, _BLK), jnp.float32),
    )(x, w1, w2)


def _route_body(nc, lg_hbm, i1_hbm, i2_hbm, i3_hbm, t1_hbm, t2_hbm, t3_hbm,
                m2_hbm, m3_hbm, k_hbm, imp_hbm,
                lg_v, i1_v, i2_v, i3_v, t1_v, t2_v, t3_v,
                m2_v, m3_v, k_v, impl_v):
    c = nc // _NW
    ng = c // 16
    wid = lax.axis_index("s") * 2 + lax.axis_index("c")
    base = wid * c
    blk = wid // (_BLK // c)
    off = (wid % (_BLK // c)) * c
    pltpu.sync_copy(lg_hbm.at[blk, :, pl.ds(off, c)], lg_v)

    zero16 = jnp.zeros((16,), jnp.float32)
    for e in range(_E):
        impl_v[e, :] = zero16

    def group(g, carry):
        tok = g * 16
        neg = jnp.full((16,), -jnp.inf, jnp.float32)
        m1, m2, m3 = neg, neg, neg
        i1 = jnp.zeros((16,), jnp.int32)
        i2, i3 = i1, i1
        for e in range(_E):
            v = lg_v[e, pl.ds(tok, 16)]
            ei = jnp.full((16,), e, jnp.int32)
            c1 = v > m1
            c2 = v > m2
            c3 = v > m3
            m3 = jnp.where(c2, m2, jnp.where(c3, v, m3))
            i3 = jnp.where(c2, i2, jnp.where(c3, ei, i3))
            m2 = jnp.where(c1, m1, jnp.where(c2, v, m2))
            i2 = jnp.where(c1, i1, jnp.where(c2, ei, i2))
            m1 = jnp.where(c1, v, m1)
            i1 = jnp.where(c1, ei, i1)
        acc = jnp.zeros((16,), jnp.float32)
        for e in range(_E):
            acc = acc + jnp.exp(lg_v[e, pl.ds(tok, 16)] - m1)
        p1 = 1.0 / acc
        p2 = jnp.exp(m2 - m1) / acc
        p3 = jnp.exp(m3 - m1) / acc
        k = jnp.where(p1 >= _P_MIN, 1,
                      jnp.where(p1 + p2 >= _P_MIN, 2, 3)).astype(jnp.int32)
        mk2 = jnp.where(k >= 2, 1.0, 0.0)
        mk3 = jnp.where(k >= 3, 1.0, 0.0)
        s = p1 + p2 * mk2 + p3 * mk3
        inv = 1.0 / (s + 1e-9)
        ts1 = p1 * inv
        ts2 = p2 * mk2 * inv
        ts3 = p3 * mk3 * inv
        sl = pl.ds(tok, 16)
        i1_v[sl] = i1
        i2_v[sl] = i2
        i3_v[sl] = i3
        t1_v[sl] = ts1
        t2_v[sl] = ts2
        t3_v[sl] = ts3
        m2_v[sl] = mk2
        m3_v[sl] = mk3
        k_v[sl] = k
        for e in range(_E):
            ei = jnp.full((16,), e, jnp.int32)
            cnt = (jnp.where(i1 == ei, ts1, 0.0)
                   + jnp.where(i2 == ei, ts2, 0.0)
                   + jnp.where(i3 == ei, ts3, 0.0))
            impl_v[e, :] = impl_v[e, :] + cnt
        return carry

    lax.fori_loop(0, ng, group, 0)

    for src, dst in ((i1_v, i1_hbm), (i2_v, i2_hbm), (i3_v, i3_hbm),
                     (t1_v, t1_hbm), (t2_v, t2_hbm), (t3_v, t3_hbm),
                     (m2_v, m2_hbm), (m3_v, m3_hbm), (k_v, k_hbm)):
        pltpu.sync_copy(src, dst.at[pl.ds(base, c)])
    pltpu.sync_copy(impl_v, imp_hbm.at[pl.ds(wid * _E, _E), :])


@functools.cache
def _make_route(nc):
    c = nc // _NW
    mesh = plsc.VectorSubcoreMesh(core_axis_name="c", subcore_axis_name="s")
    i32v = jax.ShapeDtypeStruct((nc,), jnp.int32)
    f32v = jax.ShapeDtypeStruct((nc,), jnp.float32)
    return pl.kernel(
        functools.partial(_route_body, nc),
        mesh=mesh,
        out_type=(i32v, i32v, i32v, f32v, f32v, f32v, f32v, f32v, i32v,
                  jax.ShapeDtypeStruct((_NW * _E, 16), jnp.float32)),
        scratch_types=[
            pltpu.VMEM((_E, c), jnp.float32),
            pltpu.VMEM((c,), jnp.int32),
            pltpu.VMEM((c,), jnp.int32),
            pltpu.VMEM((c,), jnp.int32),
            pltpu.VMEM((c,), jnp.float32),
            pltpu.VMEM((c,), jnp.float32),
            pltpu.VMEM((c,), jnp.float32),
            pltpu.VMEM((c,), jnp.float32),
            pltpu.VMEM((c,), jnp.float32),
            pltpu.VMEM((c,), jnp.int32),
            pltpu.VMEM((_E, 16), jnp.float32),
        ],
    )


def _loss_body(*refs):
    part_refs, loss_ref = refs[:-1], refs[-1]
    imp = jnp.zeros((_E,), jnp.float32)
    for part_ref in part_refs:
        s = jnp.sum(part_ref[...], axis=1)
        imp = imp + jnp.sum(s.reshape(_NW, _E), axis=0)
    mean = jnp.sum(imp) * (1.0 / _E)
    var = jnp.sum((imp - mean) ** 2) * (1.0 / _E)
    loss = _BAL_W * var / (mean * mean + 1e-10)
    loss_ref[...] = loss * jnp.ones((1, 1), jnp.float32)


def _loss(parts):
    return pl.pallas_call(
        _loss_body,
        out_shape=jax.ShapeDtypeStruct((1, 1), jnp.float32),
    )(*parts)


@jax.jit
def kernel(x, W1, W2):
    n, _ = x.shape
    chunks = []
    start = 0
    for nc in _CHUNKS:
        logits_t = _mlp_logits_t(x, W1, W2, start // _BLK, nc // _BLK)
        chunks.append(_make_route(nc)(logits_t))
        start += nc
    i1, i2, i3, t1, t2, t3, m2, m3, k_vec = (
        jnp.concatenate([ch[j] for ch in chunks]) if len(chunks) > 1
        else chunks[0][j] for j in range(9))
    loss = _loss([ch[9] for ch in chunks])
    top_idx = jnp.stack([i1, i2, i3], axis=1)
    top_scores = jnp.stack([t1, t2, t3], axis=1)
    top_mask = jnp.stack([jnp.ones((n,), jnp.float32), m2, m3], axis=1)
    return (top_idx, top_scores, top_mask, k_vec, loss.reshape(()))


# speedup vs baseline: 1.0408x; 1.0021x over previous
import functools

import jax
import jax.numpy as jnp
from jax import lax
from jax.experimental import pallas as pl
from jax.experimental.pallas import tpu as pltpu
from jax.experimental.pallas import tpu_sc as plsc

_E = 64
_TEMP = 0.7
_P_MIN = 0.92
_K = 3
_BAL_W = 0.01
_BLK = 1024
_N = 16384
_NW = 32
_CHUNKS = (8192, 8192)


def _mlp_body(x_ref, w1_ref, w2_ref, o_ref):
    h = jnp.tanh(lax.dot_general(
        x_ref[...], w1_ref[...], (((1,), (1,)), ((), ())),
        preferred_element_type=jnp.float32))
    o_ref[0] = lax.dot_general(
        w2_ref[...], h, (((1,), (1,)), ((), ())),
        preferred_element_type=jnp.float32) * (1.0 / _TEMP)


def _mlp_logits_t(x, w1, w2, start_blk, nb):
    n, d = x.shape
    return pl.pallas_call(
        _mlp_body,
        grid=(nb,),
        in_specs=[
            pl.BlockSpec((_BLK, d), lambda i: (i + start_blk, 0)),
            pl.BlockSpec((_E, d), lambda i: (0, 0)),
            pl.BlockSpec((_E, _E), lambda i: (0, 0)),
        ],
        out_specs=pl.BlockSpec((1, _E, _BLK), lambda i: (i, 0, 0)),
        out_shape=jax.ShapeDtypeStruct((nb, _E, _BLK), jnp.float32),
    )(x, w1, w2)


def _route_body(nc, lg_hbm, i1_hbm, i2_hbm, i3_hbm, t1_hbm, t2_hbm, t3_hbm,
                m2_hbm, m3_hbm, k_hbm, imp_hbm,
                lg_v, i1_v, i2_v, i3_v, t1_v, t2_v, t3_v,
                m2_v, m3_v, k_v, impl_v):
    c = nc // _NW
    ng = c // 16
    wid = lax.axis_index("s") * 2 + lax.axis_index("c")
    base = wid * c
    blk = wid // (_BLK // c)
    off = (wid % (_BLK // c)) * c
    pltpu.sync_copy(lg_hbm.at[blk, :, pl.ds(off, c)], lg_v)

    zero16 = jnp.zeros((16,), jnp.float32)
    for e in range(_E):
        impl_v[e, :] = zero16

    def group(g, carry):
        tok = g * 16
        neg = jnp.full((16,), -jnp.inf, jnp.float32)
        m1, m2, m3 = neg, neg, neg
        i1 = jnp.zeros((16,), jnp.int32)
        i2, i3 = i1, i1
        for e in range(_E):
            v = lg_v[e, pl.ds(tok, 16)]
            ei = jnp.full((16,), e, jnp.int32)
            c1 = v > m1
            c2 = v > m2
            c3 = v > m3
            m3 = jnp.where(c2, m2, jnp.where(c3, v, m3))
            i3 = jnp.where(c2, i2, jnp.where(c3, ei, i3))
            m2 = jnp.where(c1, m1, jnp.where(c2, v, m2))
            i2 = jnp.where(c1, i1, jnp.where(c2, ei, i2))
            m1 = jnp.where(c1, v, m1)
            i1 = jnp.where(c1, ei, i1)
        acc = jnp.zeros((16,), jnp.float32)
        for e in range(_E):
            acc = acc + jnp.exp(lg_v[e, pl.ds(tok, 16)] - m1)
        p1 = 1.0 / acc
        p2 = jnp.exp(m2 - m1) / acc
        p3 = jnp.exp(m3 - m1) / acc
        k = jnp.where(p1 >= _P_MIN, 1,
                      jnp.where(p1 + p2 >= _P_MIN, 2, 3)).astype(jnp.int32)
        mk2 = jnp.where(k >= 2, 1.0, 0.0)
        mk3 = jnp.where(k >= 3, 1.0, 0.0)
        s = p1 + p2 * mk2 + p3 * mk3
        inv = 1.0 / (s + 1e-9)
        ts1 = p1 * inv
        ts2 = p2 * mk2 * inv
        ts3 = p3 * mk3 * inv
        sl = pl.ds(tok, 16)
        i1_v[sl] = i1
        i2_v[sl] = i2
        i3_v[sl] = i3
        t1_v[sl] = ts1
        t2_v[sl] = ts2
        t3_v[sl] = ts3
        m2_v[sl] = mk2
        m3_v[sl] = mk3
        k_v[sl] = k
        for e in range(_E):
            ei = jnp.full((16,), e, jnp.int32)
            cnt = (jnp.where(i1 == ei, ts1, 0.0)
                   + jnp.where(i2 == ei, ts2, 0.0)
                   + jnp.where(i3 == ei, ts3, 0.0))
            impl_v[e, :] = impl_v[e, :] + cnt
        return carry

    lax.fori_loop(0, ng, group, 0)

    for src, dst in ((i1_v, i1_hbm), (i2_v, i2_hbm), (i3_v, i3_hbm),
                     (t1_v, t1_hbm), (t2_v, t2_hbm), (t3_v, t3_hbm),
                     (m2_v, m2_hbm), (m3_v, m3_hbm), (k_v, k_hbm)):
        pltpu.sync_copy(src, dst.at[pl.ds(base, c)])
    pltpu.sync_copy(impl_v, imp_hbm.at[pl.ds(wid * _E, _E), :])


@functools.cache
def _make_route(nc):
    c = nc // _NW
    mesh = plsc.VectorSubcoreMesh(core_axis_name="c", subcore_axis_name="s")
    i32v = jax.ShapeDtypeStruct((nc,), jnp.int32)
    f32v = jax.ShapeDtypeStruct((nc,), jnp.float32)
    return pl.kernel(
        functools.partial(_route_body, nc),
        mesh=mesh,
        out_type=(i32v, i32v, i32v, f32v, f32v, f32v, f32v, f32v, i32v,
                  jax.ShapeDtypeStruct((_NW * _E, 16), jnp.float32)),
        scratch_types=[
            pltpu.VMEM((_E, c), jnp.float32),
            pltpu.VMEM((c,), jnp.int32),
            pltpu.VMEM((c,), jnp.int32),
            pltpu.VMEM((c,), jnp.int32),
            pltpu.VMEM((c,), jnp.float32),
            pltpu.VMEM((c,), jnp.float32),
            pltpu.VMEM((c,), jnp.float32),
            pltpu.VMEM((c,), jnp.float32),
            pltpu.VMEM((c,), jnp.float32),
            pltpu.VMEM((c,), jnp.int32),
            pltpu.VMEM((_E, 16), jnp.float32),
        ],
    )


def _loss_body(*refs):
    part_refs, loss_ref = refs[:-1], refs[-1]
    imp = jnp.zeros((_E,), jnp.float32)
    for part_ref in part_refs:
        s = jnp.sum(part_ref[...], axis=1)
        imp = imp + jnp.sum(s.reshape(_NW, _E), axis=0)
    mean = jnp.sum(imp) * (1.0 / _E)
    var = jnp.sum((imp - mean) ** 2) * (1.0 / _E)
    loss = _BAL_W * var / (mean * mean + 1e-10)
    loss_ref[...] = loss * jnp.ones((1, 1), jnp.float32)


def _loss(parts):
    return pl.pallas_call(
        _loss_body,
        out_shape=jax.ShapeDtypeStruct((1, 1), jnp.float32),
    )(*parts)


@jax.jit
def kernel(x, W1, W2):
    n, _ = x.shape
    chunks = []
    start = 0
    for nc in _CHUNKS:
        logits_t = _mlp_logits_t(x, W1, W2, start // _BLK, nc // _BLK)
        chunks.append(_make_route(nc)(logits_t))
        start += nc
    loss = _loss([ch[9] for ch in chunks])
    idx_c, ts_c, mask_c = [], [], []
    for nc, ch in zip(_CHUNKS, chunks):
        i1, i2, i3, t1, t2, t3, m2, m3 = ch[:8]
        idx_c.append(jnp.stack([i1, i2, i3], axis=1))
        ts_c.append(jnp.stack([t1, t2, t3], axis=1))
        mask_c.append(jnp.stack([jnp.ones((nc,), jnp.float32), m2, m3],
                                axis=1))
    top_idx = jnp.concatenate(idx_c)
    top_scores = jnp.concatenate(ts_c)
    top_mask = jnp.concatenate(mask_c)
    k_vec = jnp.concatenate([ch[8] for ch in chunks])
    return (top_idx, top_scores, top_mask, k_vec, loss.reshape(()))
